# baseline (speedup 1.0000x reference)
import functools

import jax
import jax.numpy as jnp
from jax import lax
from jax.experimental import pallas as pl
from jax.experimental.pallas import tpu as pltpu
from jax.experimental.pallas import tpu_sc as plsc

D = 128
B = 16384
NW = 32
ROWS_W = B // NW
CHUNK = 128
NCHUNK = ROWS_W // CHUNK
L = 16
GROUPS = CHUNK // L


def _rsqrt_nr(x):
    i = plsc.bitcast(x, jnp.int32)
    y = plsc.bitcast(jnp.int32(0x5F3759DF) - (i >> 1), jnp.float32)
    for _ in range(3):
        y = y * (jnp.float32(1.5) - jnp.float32(0.5) * x * y * y)
    return y


def _body(ent_e, rel_e, ent_t, rel_t, lidx, ridx, hidx, out,
          lidx_v, ridx_v, hidx_v, bA, bB, bTl, bTh, bR, bRt, out_v, sem):
    wid = lax.axis_index("s") * 2 + lax.axis_index("c")
    base = wid * ROWS_W
    for c in range(NCHUNK):
        cbase = base + c * CHUNK
        pltpu.sync_copy(lidx.at[pl.ds(cbase, CHUNK)], lidx_v)
        pltpu.sync_copy(ridx.at[pl.ds(cbase, CHUNK)], ridx_v)
        pltpu.sync_copy(hidx.at[pl.ds(cbase, CHUNK)], hidx_v)
        cps = [
            pltpu.async_copy(ent_e.at[lidx_v], bA, sem),
            pltpu.async_copy(ent_e.at[hidx_v], bB, sem),
            pltpu.async_copy(ent_t.at[lidx_v], bTl, sem),
            pltpu.async_copy(ent_t.at[hidx_v], bTh, sem),
            pltpu.async_copy(rel_e.at[ridx_v], bR, sem),
            pltpu.async_copy(rel_t.at[ridx_v], bRt, sem),
        ]
        for cp in cps:
            cp.wait()

        def group(g, carry):
            rows = g * L + lax.iota(jnp.int32, L)
            zeros = jnp.zeros((L,), jnp.float32)

            def dstep(dd, acc):
                cols = jnp.full((L,), dd, jnp.int32)
                a = plsc.load_gather(bA, [rows, cols])
                b = plsc.load_gather(bB, [rows, cols])
                tl = plsc.load_gather(bTl, [rows, cols])
                th = plsc.load_gather(bTh, [rows, cols])
                r = plsc.load_gather(bR, [rows, cols])
                rt = plsc.load_gather(bRt, [rows, cols])
                (ssA, ssB, ssTl, ssTh, ssR, ssRt,
                 dATl, dBTh, dAB, dAR, dARt, dBR, dBRt, dRRt) = acc
                return (ssA + a * a, ssB + b * b, ssTl + tl * tl,
                        ssTh + th * th, ssR + r * r, ssRt + rt * rt,
                        dATl + a * tl, dBTh + b * th, dAB + a * b,
                        dAR + a * r, dARt + a * rt, dBR + b * r,
                        dBRt + b * rt, dRRt + r * rt)

            (ssA, ssB, ssTl, ssTh, ssR, ssRt,
             dATl, dBTh, dAB, dAR, dARt, dBR, dBRt, dRRt) = lax.fori_loop(
                 0, D, dstep, (zeros,) * 14)

            one = jnp.float32(1.0)
            sA = jnp.minimum(one, _rsqrt_nr(ssA))
            sB = jnp.minimum(one, _rsqrt_nr(ssB))
            sTl = jnp.minimum(one, _rsqrt_nr(ssTl))
            sTh = jnp.minimum(one, _rsqrt_nr(ssTh))
            sR = jnp.minimum(one, _rsqrt_nr(ssR))
            sRt = jnp.minimum(one, _rsqrt_nr(ssRt))
            w = (sA * sTl * dATl - sB * sTh * dBTh) * sRt
            ssd = (sA * sA * ssA + sB * sB * ssB + sR * sR * ssR
                   + w * w * ssRt
                   + jnp.float32(2.0) * (sA * sR * dAR - sA * sB * dAB
                                         + sA * w * dARt - sB * sR * dBR
                                         - sB * w * dBRt + sR * w * dRRt))
            ssd = jnp.maximum(ssd, jnp.float32(0.0))
            enrg = ssd * _rsqrt_nr(ssd)
            out_v[pl.ds(c * CHUNK + g * L, L)] = enrg
            return carry

        lax.fori_loop(0, GROUPS, group, jnp.int32(0))
    pltpu.sync_copy(out_v, out.at[pl.ds(base, ROWS_W)])


_sc_call = functools.partial(
    pl.kernel,
    out_type=jax.ShapeDtypeStruct((B,), jnp.float32),
    mesh=plsc.VectorSubcoreMesh(core_axis_name="c", subcore_axis_name="s"),
    compiler_params=pltpu.CompilerParams(use_tc_tiling_on_sc=False, needs_layout_passes=False),
    scratch_types=[
        pltpu.VMEM((CHUNK,), jnp.int32),
        pltpu.VMEM((CHUNK,), jnp.int32),
        pltpu.VMEM((CHUNK,), jnp.int32),
        pltpu.VMEM((CHUNK, D), jnp.float32),
        pltpu.VMEM((CHUNK, D), jnp.float32),
        pltpu.VMEM((CHUNK, D), jnp.float32),
        pltpu.VMEM((CHUNK, D), jnp.float32),
        pltpu.VMEM((CHUNK, D), jnp.float32),
        pltpu.VMEM((CHUNK, D), jnp.float32),
        pltpu.VMEM((ROWS_W,), jnp.float32),
        pltpu.SemaphoreType.DMA,
    ],
)


@jax.jit
def kernel(ent_embeds, rel_embeds, ent_transfer, rel_transfer, triplets):
    t = triplets.astype(jnp.int32)
    lidx = t[:, 0]
    ridx = t[:, 1]
    hidx = t[:, 2]
    return _sc_call(_body)(ent_embeds, rel_embeds, ent_transfer, rel_transfer,
                           lidx, ridx, hidx)


# speedup vs baseline: 3.4471x; 3.4471x over previous
import functools

import jax
import jax.numpy as jnp
from jax import lax
from jax.experimental import pallas as pl
from jax.experimental.pallas import tpu as pltpu
from jax.experimental.pallas import tpu_sc as plsc

D = 128
B = 16384
NW = 32
ROWS_W = B // NW
CHUNK = 128
NCHUNK = ROWS_W // CHUNK
L = 16
GROUPS = CHUNK // L


def _rsqrt_nr(x):
    i = plsc.bitcast(x, jnp.int32)
    y = plsc.bitcast(jnp.int32(0x5F3759DF) - (i >> 1), jnp.float32)
    for _ in range(3):
        y = y * (jnp.float32(1.5) - jnp.float32(0.5) * x * y * y)
    return y


def _body(ent_e, rel_e, ent_t, rel_t, lidx, ridx, hidx, out,
          lidx_v, ridx_v, hidx_v, bA, bB, bTl, bTh, bR, bRt, stg, out_v, sem):
    wid = lax.axis_index("s") * 2 + lax.axis_index("c")
    base = wid * ROWS_W
    lastlane = lax.iota(jnp.int32, L) == jnp.int32(L - 1)
    for c in range(NCHUNK):
        cbase = base + c * CHUNK
        pltpu.sync_copy(lidx.at[pl.ds(cbase, CHUNK)], lidx_v)
        pltpu.sync_copy(ridx.at[pl.ds(cbase, CHUNK)], ridx_v)
        pltpu.sync_copy(hidx.at[pl.ds(cbase, CHUNK)], hidx_v)
        cps = [
            pltpu.async_copy(ent_e.at[lidx_v], bA, sem),
            pltpu.async_copy(ent_e.at[hidx_v], bB, sem),
            pltpu.async_copy(ent_t.at[lidx_v], bTl, sem),
            pltpu.async_copy(ent_t.at[hidx_v], bTh, sem),
            pltpu.async_copy(rel_e.at[ridx_v], bR, sem),
            pltpu.async_copy(rel_t.at[ridx_v], bRt, sem),
        ]
        for cp in cps:
            cp.wait()

        def group(g, carry):
            def rowfn(r, rcarry):
                row = g * L + r
                prods = None
                for k in range(8):
                    sl = pl.ds(k * L, L)
                    a = bA[row, sl]
                    b = bB[row, sl]
                    tl = bTl[row, sl]
                    th = bTh[row, sl]
                    rr = bR[row, sl]
                    rt = bRt[row, sl]
                    terms = (a * a, b * b, tl * tl, th * th, rr * rr,
                             rt * rt, a * tl, b * th, a * b, a * rr,
                             a * rt, b * rr, b * rt, rr * rt)
                    if prods is None:
                        prods = list(terms)
                    else:
                        prods = [p + t for p, t in zip(prods, terms)]
                for q in range(14):
                    cs = plsc.cumsum(prods[q])
                    plsc.store_scatter(
                        stg, [jnp.full((L,), q * L, jnp.int32) + r], cs,
                        mask=lastlane)
                return rcarry

            lax.fori_loop(0, L, rowfn, jnp.int32(0))

            (ssA, ssB, ssTl, ssTh, ssR, ssRt,
             dATl, dBTh, dAB, dAR, dARt, dBR, dBRt, dRRt) = [
                 stg[pl.ds(q * L, L)] for q in range(14)]

            one = jnp.float32(1.0)
            sA = jnp.minimum(one, _rsqrt_nr(ssA))
            sB = jnp.minimum(one, _rsqrt_nr(ssB))
            sTl = jnp.minimum(one, _rsqrt_nr(ssTl))
            sTh = jnp.minimum(one, _rsqrt_nr(ssTh))
            sR = jnp.minimum(one, _rsqrt_nr(ssR))
            sRt = jnp.minimum(one, _rsqrt_nr(ssRt))
            w = (sA * sTl * dATl - sB * sTh * dBTh) * sRt
            ssd = (sA * sA * ssA + sB * sB * ssB + sR * sR * ssR
                   + w * w * ssRt
                   + jnp.float32(2.0) * (sA * sR * dAR - sA * sB * dAB
                                         + sA * w * dARt - sB * sR * dBR
                                         - sB * w * dBRt + sR * w * dRRt))
            ssd = jnp.maximum(ssd, jnp.float32(0.0))
            enrg = ssd * _rsqrt_nr(ssd)
            out_v[pl.ds(c * CHUNK + g * L, L)] = enrg
            return carry

        lax.fori_loop(0, GROUPS, group, jnp.int32(0))
    pltpu.sync_copy(out_v, out.at[pl.ds(base, ROWS_W)])


_sc_call = functools.partial(
    pl.kernel,
    out_type=jax.ShapeDtypeStruct((B,), jnp.float32),
    mesh=plsc.VectorSubcoreMesh(core_axis_name="c", subcore_axis_name="s"),
    compiler_params=pltpu.CompilerParams(use_tc_tiling_on_sc=False, needs_layout_passes=False),
    scratch_types=[
        pltpu.VMEM((CHUNK,), jnp.int32),
        pltpu.VMEM((CHUNK,), jnp.int32),
        pltpu.VMEM((CHUNK,), jnp.int32),
        pltpu.VMEM((CHUNK, D), jnp.float32),
        pltpu.VMEM((CHUNK, D), jnp.float32),
        pltpu.VMEM((CHUNK, D), jnp.float32),
        pltpu.VMEM((CHUNK, D), jnp.float32),
        pltpu.VMEM((CHUNK, D), jnp.float32),
        pltpu.VMEM((CHUNK, D), jnp.float32),
        pltpu.VMEM((14 * L,), jnp.float32),
        pltpu.VMEM((ROWS_W,), jnp.float32),
        pltpu.SemaphoreType.DMA,
    ],
)


@jax.jit
def kernel(ent_embeds, rel_embeds, ent_transfer, rel_transfer, triplets):
    t = triplets.astype(jnp.int32)
    lidx = t[:, 0]
    ridx = t[:, 1]
    hidx = t[:, 2]
    return _sc_call(_body)(ent_embeds, rel_embeds, ent_transfer, rel_transfer,
                           lidx, ridx, hidx)


# speedup vs baseline: 4.5571x; 1.3220x over previous
import functools

import jax
import jax.numpy as jnp
from jax import lax
from jax.experimental import pallas as pl
from jax.experimental.pallas import tpu as pltpu
from jax.experimental.pallas import tpu_sc as plsc

D = 128
B = 16384
NW = 32
ROWS_W = B // NW
CHUNK = 64
NCHUNK = ROWS_W // CHUNK
L = 16
GROUPS = CHUNK // L


def _rsqrt_nr(x):
    i = plsc.bitcast(x, jnp.int32)
    y = plsc.bitcast(jnp.int32(0x5F3759DF) - (i >> 1), jnp.float32)
    for _ in range(3):
        y = y * (jnp.float32(1.5) - jnp.float32(0.5) * x * y * y)
    return y


def _body(ent_e, rel_e, ent_t, rel_t, lidx, ridx, hidx, out,
          lidx_v, ridx_v, hidx_v,
          bA0, bB0, bTl0, bTh0, bR0, bRt0,
          bA1, bB1, bTl1, bTh1, bR1, bRt1,
          stg, out_v, sem0, sem1):
    wid = lax.axis_index("s") * 2 + lax.axis_index("c")
    base = wid * ROWS_W
    lastlane = lax.iota(jnp.int32, L) == jnp.int32(L - 1)
    pltpu.sync_copy(lidx.at[pl.ds(base, ROWS_W)], lidx_v)
    pltpu.sync_copy(ridx.at[pl.ds(base, ROWS_W)], ridx_v)
    pltpu.sync_copy(hidx.at[pl.ds(base, ROWS_W)], hidx_v)

    bufs = [(bA0, bB0, bTl0, bTh0, bR0, bRt0),
            (bA1, bB1, bTl1, bTh1, bR1, bRt1)]
    sems = [sem0, sem1]

    def issue(c):
        bA, bB, bTl, bTh, bR, bRt = bufs[c % 2]
        sm = sems[c % 2]
        ls = lidx_v.at[pl.ds(c * CHUNK, CHUNK)]
        rs = ridx_v.at[pl.ds(c * CHUNK, CHUNK)]
        hs = hidx_v.at[pl.ds(c * CHUNK, CHUNK)]
        return [pltpu.async_copy(ent_e.at[ls], bA, sm),
                pltpu.async_copy(ent_e.at[hs], bB, sm),
                pltpu.async_copy(ent_t.at[ls], bTl, sm),
                pltpu.async_copy(ent_t.at[hs], bTh, sm),
                pltpu.async_copy(rel_e.at[rs], bR, sm),
                pltpu.async_copy(rel_t.at[rs], bRt, sm)]

    def compute(c):
        bA, bB, bTl, bTh, bR, bRt = bufs[c % 2]

        def group(g, carry):
            def rowfn(r, rcarry):
                row = g * L + r
                prods = None
                for k in range(8):
                    sl = pl.ds(k * L, L)
                    a = bA[row, sl]
                    b = bB[row, sl]
                    tl = bTl[row, sl]
                    th = bTh[row, sl]
                    rr = bR[row, sl]
                    rt = bRt[row, sl]
                    terms = (a * a, b * b, tl * tl, th * th, rr * rr,
                             rt * rt, a * tl, b * th, a * b, a * rr,
                             a * rt, b * rr, b * rt, rr * rt)
                    if prods is None:
                        prods = list(terms)
                    else:
                        prods = [p + t for p, t in zip(prods, terms)]
                for q in range(14):
                    cs = plsc.cumsum(prods[q])
                    plsc.store_scatter(
                        stg, [jnp.full((L,), q * L, jnp.int32) + r], cs,
                        mask=lastlane)
                return rcarry

            lax.fori_loop(0, L, rowfn, jnp.int32(0))

            (ssA, ssB, ssTl, ssTh, ssR, ssRt,
             dATl, dBTh, dAB, dAR, dARt, dBR, dBRt, dRRt) = [
                 stg[pl.ds(q * L, L)] for q in range(14)]

            one = jnp.float32(1.0)
            sA = jnp.minimum(one, _rsqrt_nr(ssA))
            sB = jnp.minimum(one, _rsqrt_nr(ssB))
            sTl = jnp.minimum(one, _rsqrt_nr(ssTl))
            sTh = jnp.minimum(one, _rsqrt_nr(ssTh))
            sR = jnp.minimum(one, _rsqrt_nr(ssR))
            sRt = jnp.minimum(one, _rsqrt_nr(ssRt))
            w = (sA * sTl * dATl - sB * sTh * dBTh) * sRt
            ssd = (sA * sA * ssA + sB * sB * ssB + sR * sR * ssR
                   + w * w * ssRt
                   + jnp.float32(2.0) * (sA * sR * dAR - sA * sB * dAB
                                         + sA * w * dARt - sB * sR * dBR
                                         - sB * w * dBRt + sR * w * dRRt))
            ssd = jnp.maximum(ssd, jnp.float32(0.0))
            enrg = ssd * _rsqrt_nr(ssd)
            out_v[pl.ds(c * CHUNK + g * L, L)] = enrg
            return carry

        lax.fori_loop(0, GROUPS, group, jnp.int32(0))

    pending = issue(0)
    for c in range(NCHUNK):
        nxt = issue(c + 1) if c + 1 < NCHUNK else None
        for cp in pending:
            cp.wait()
        compute(c)
        pending = nxt
    pltpu.sync_copy(out_v, out.at[pl.ds(base, ROWS_W)])


_sc_call = functools.partial(
    pl.kernel,
    out_type=jax.ShapeDtypeStruct((B,), jnp.float32),
    mesh=plsc.VectorSubcoreMesh(core_axis_name="c", subcore_axis_name="s"),
    compiler_params=pltpu.CompilerParams(use_tc_tiling_on_sc=False,
                                         needs_layout_passes=False),
    scratch_types=(
        [pltpu.VMEM((ROWS_W,), jnp.int32)] * 3
        + [pltpu.VMEM((CHUNK, D), jnp.float32)] * 12
        + [pltpu.VMEM((14 * L,), jnp.float32),
           pltpu.VMEM((ROWS_W,), jnp.float32),
           pltpu.SemaphoreType.DMA,
           pltpu.SemaphoreType.DMA]
    ),
)


@jax.jit
def kernel(ent_embeds, rel_embeds, ent_transfer, rel_transfer, triplets):
    t = triplets.astype(jnp.int32)
    lidx = t[:, 0]
    ridx = t[:, 1]
    hidx = t[:, 2]
    return _sc_call(_body)(ent_embeds, rel_embeds, ent_transfer, rel_transfer,
                           lidx, ridx, hidx)
